# baseline (speedup 1.0000x reference)
import functools

import jax
import jax.numpy as jnp
from jax import lax
from jax.experimental import pallas as pl
from jax.experimental.pallas import tpu as pltpu
from jax.experimental.pallas import tpu_sc as plsc

HID = 128
SEQ = 200
N_TOK = 1024 * 200
NC, NS = 2, 16
NW = NC * NS
CHUNK = N_TOK // NW
T = 64
NSTEPS = CHUNK // T
EPS = 1e-12


def _body(ids_h, cids_h, rids_h, word_h, pos_h, type_h, col_h, row_h,
          gam_h, bet_h, out_h,
          widx_v, cidx_v, ridx_v, base_v, colt_v, rowt_v,
          typ_v, gam_v, bet_v, wbuf0, wbuf1, obuf0, obuf1,
          gsem0, gsem1, osem0, osem1):
  wid = lax.axis_index("s") * NC + lax.axis_index("c")
  tok0 = wid * CHUNK

  pltpu.sync_copy(ids_h.at[pl.ds(tok0, CHUNK)], widx_v)
  pltpu.sync_copy(cids_h.at[pl.ds(tok0, CHUNK)], cidx_v)
  pltpu.sync_copy(rids_h.at[pl.ds(tok0, CHUNK)], ridx_v)
  pltpu.sync_copy(pos_h.at[pl.ds(0, SEQ * HID)], base_v)
  pltpu.sync_copy(col_h, colt_v)
  pltpu.sync_copy(row_h, rowt_v)
  pltpu.sync_copy(type_h, typ_v)
  pltpu.sync_copy(gam_h, gam_v)
  pltpu.sync_copy(bet_h, bet_v)

  tv = [typ_v[pl.ds(j * 16, 16)] for j in range(8)]

  def fold_type(s, c):
    for j in range(8):
      off = s * HID + j * 16
      base_v[pl.ds(off, 16)] = base_v[pl.ds(off, 16)] + tv[j]
    return c
  lax.fori_loop(0, SEQ, fold_type, 0)

  gv = [gam_v[pl.ds(j * 16, 16)] for j in range(8)]
  bv = [bet_v[pl.ds(j * 16, 16)] for j in range(8)]
  iot = lax.iota(jnp.int32, 16)

  def group_body(i, wbuf, obuf, g, c):
    gbase = i * T + g * 16
    civ = cidx_v[pl.ds(gbase, 16)]
    riv = ridx_v[pl.ds(gbase, 16)]
    for k in range(16):
      t = g * 16 + k
      s = lax.rem(gbase + k, SEQ)
      cb = civ[k] * HID
      rb = riv[k] * HID
      sb = s * HID
      xs = []
      for j in range(8):
        w = wbuf[t, pl.ds(j * 16, 16)]
        b = base_v[pl.ds(sb + j * 16, 16)]
        cvec = plsc.load_gather(colt_v, [cb + j * 16 + iot])
        rvec = plsc.load_gather(rowt_v, [rb + j * 16 + iot])
        xs.append((w + b) + (cvec + rvec))
      acc = ((xs[0] + xs[1]) + (xs[2] + xs[3])) + ((xs[4] + xs[5]) + (xs[6] + xs[7]))
      mean = jnp.sum(acc) * (1.0 / HID)
      cs = [x - mean for x in xs]
      sq = (((cs[0] * cs[0] + cs[1] * cs[1]) + (cs[2] * cs[2] + cs[3] * cs[3]))
            + ((cs[4] * cs[4] + cs[5] * cs[5]) + (cs[6] * cs[6] + cs[7] * cs[7])))
      var = jnp.sum(sq) * (1.0 / HID)
      vv = jnp.broadcast_to(var + EPS, (16,))
      bi = plsc.bitcast(vv, jnp.int32)
      y = plsc.bitcast(jnp.int32(0x5F3759DF) - lax.shift_right_arithmetic(bi, 1),
                       jnp.float32)
      for _ in range(3):
        y = y * (1.5 - 0.5 * vv * y * y)
      for j in range(8):
        obuf[t, pl.ds(j * 16, 16)] = (cs[j] * y) * gv[j] + bv[j]
    return c

  def gather(i, buf, sem):
    return pltpu.make_async_copy(word_h.at[widx_v.at[pl.ds(i * T, T)]],
                                 buf, sem)

  def outcp(i, buf, sem):
    return pltpu.make_async_copy(buf, out_h.at[pl.ds(tok0 + i * T, T)], sem)

  def compute(i, wbuf, obuf):
    lax.fori_loop(0, T // 16, functools.partial(group_body, i, wbuf, obuf), 0)

  gather(0, wbuf0, gsem0).start()

  def pair(g, c):
    i0, i1 = 2 * g, 2 * g + 1
    gather(i1, wbuf1, gsem1).start()
    gather(i0, wbuf0, gsem0).wait()
    pl.when(g > 0)(lambda: outcp(i0, obuf0, osem0).wait())
    compute(i0, wbuf0, obuf0)
    outcp(i0, obuf0, osem0).start()
    pl.when(i1 + 1 < NSTEPS)(lambda: gather(i1 + 1, wbuf0, gsem0).start())
    gather(i1, wbuf1, gsem1).wait()
    pl.when(g > 0)(lambda: outcp(i1, obuf1, osem1).wait())
    compute(i1, wbuf1, obuf1)
    outcp(i1, obuf1, osem1).start()
    return c

  lax.fori_loop(0, NSTEPS // 2, pair, 0)
  outcp(NSTEPS - 2, obuf0, osem0).wait()
  outcp(NSTEPS - 1, obuf1, osem1).wait()


_emb = functools.partial(
    pl.kernel,
    out_type=jax.ShapeDtypeStruct((N_TOK, HID), jnp.float32),
    mesh=plsc.VectorSubcoreMesh(core_axis_name="c", subcore_axis_name="s",
                                num_cores=NC, num_subcores=NS),
    compiler_params=pltpu.CompilerParams(needs_layout_passes=False),
    scratch_types=[
        pltpu.VMEM((CHUNK,), jnp.int32),
        pltpu.VMEM((CHUNK,), jnp.int32),
        pltpu.VMEM((CHUNK,), jnp.int32),
        pltpu.VMEM((SEQ * HID,), jnp.float32),
        pltpu.VMEM((51 * HID,), jnp.float32),
        pltpu.VMEM((101 * HID,), jnp.float32),
        pltpu.VMEM((HID,), jnp.float32),
        pltpu.VMEM((HID,), jnp.float32),
        pltpu.VMEM((HID,), jnp.float32),
        pltpu.VMEM((T, HID), jnp.float32),
        pltpu.VMEM((T, HID), jnp.float32),
        pltpu.VMEM((T, HID), jnp.float32),
        pltpu.VMEM((T, HID), jnp.float32),
        pltpu.SemaphoreType.DMA,
        pltpu.SemaphoreType.DMA,
        pltpu.SemaphoreType.DMA,
        pltpu.SemaphoreType.DMA,
    ],
)(_body)


def kernel(input_ids, column_ids, row_ids, word_emb, pos_emb, type_emb,
           col_emb, row_emb, ln_gamma, ln_beta):
  bsz, seq_len = input_ids.shape
  ids = input_ids.reshape(-1).astype(jnp.int32)
  cids = column_ids.reshape(-1).astype(jnp.int32)
  rids = row_ids.reshape(-1).astype(jnp.int32)
  out = _emb(ids, cids, rids, word_emb, pos_emb.reshape(-1),
             type_emb[0], col_emb.reshape(-1), row_emb.reshape(-1),
             ln_gamma, ln_beta)
  return out.reshape(bsz, seq_len, HID)


# speedup vs baseline: 1.0062x; 1.0062x over previous
import functools

import jax
import jax.numpy as jnp
from jax import lax
from jax.experimental import pallas as pl
from jax.experimental.pallas import tpu as pltpu
from jax.experimental.pallas import tpu_sc as plsc

HID = 128
SEQ = 200
N_TOK = 1024 * 200
NC, NS = 2, 16
NW = NC * NS
CHUNK = N_TOK // NW
T = 64
NSTEPS = CHUNK // T
EPS = 1e-12


def _body(ids_h, cids_h, rids_h, word_h, pos_h, type_h, col_h, row_h,
          gam_h, bet_h, out_h,
          widx_v, cidx_v, ridx_v, base_v, colt_v, rowt_v,
          typ_v, gam_v, bet_v, wbuf0, wbuf1, obuf0, obuf1,
          gsem0, gsem1, osem0, osem1):
  wid = lax.axis_index("s") * NC + lax.axis_index("c")
  tok0 = wid * CHUNK

  pltpu.sync_copy(ids_h.at[pl.ds(tok0, CHUNK)], widx_v)
  pltpu.sync_copy(cids_h.at[pl.ds(tok0, CHUNK)], cidx_v)
  pltpu.sync_copy(rids_h.at[pl.ds(tok0, CHUNK)], ridx_v)
  pltpu.sync_copy(pos_h.at[pl.ds(0, SEQ * HID)], base_v)
  pltpu.sync_copy(col_h, colt_v)
  pltpu.sync_copy(row_h, rowt_v)
  pltpu.sync_copy(type_h, typ_v)
  pltpu.sync_copy(gam_h, gam_v)
  pltpu.sync_copy(bet_h, bet_v)

  tv = [typ_v[pl.ds(j * 16, 16)] for j in range(8)]

  def fold_type(s, c):
    for j in range(8):
      off = s * HID + j * 16
      base_v[pl.ds(off, 16)] = base_v[pl.ds(off, 16)] + tv[j]
    return c
  lax.fori_loop(0, SEQ, fold_type, 0)

  gv = [gam_v[pl.ds(j * 16, 16)] for j in range(8)]
  bv = [bet_v[pl.ds(j * 16, 16)] for j in range(8)]
  iot = lax.iota(jnp.int32, 16)

  def group_body(i, wbuf, obuf, g, c):
    gbase = i * T + g * 16
    civ = cidx_v[pl.ds(gbase, 16)]
    riv = ridx_v[pl.ds(gbase, 16)]
    for k in range(16):
      t = g * 16 + k
      s = lax.rem(gbase + k, SEQ)
      cb = civ[k] * HID
      rb = riv[k] * HID
      sb = s * HID
      xs = []
      for j in range(8):
        w = wbuf[t, pl.ds(j * 16, 16)]
        b = base_v[pl.ds(sb + j * 16, 16)]
        cvec = plsc.load_gather(colt_v, [cb + j * 16 + iot])
        rvec = plsc.load_gather(rowt_v, [rb + j * 16 + iot])
        xs.append((w + b) + (cvec + rvec))
      acc = ((xs[0] + xs[1]) + (xs[2] + xs[3])) + ((xs[4] + xs[5]) + (xs[6] + xs[7]))
      mean = jnp.sum(acc) * (1.0 / HID)
      cs = [x - mean for x in xs]
      sq = (((cs[0] * cs[0] + cs[1] * cs[1]) + (cs[2] * cs[2] + cs[3] * cs[3]))
            + ((cs[4] * cs[4] + cs[5] * cs[5]) + (cs[6] * cs[6] + cs[7] * cs[7])))
      var = jnp.sum(sq) * (1.0 / HID)
      vv = jnp.broadcast_to(var + EPS, (16,))
      bi = plsc.bitcast(vv, jnp.int32)
      y = plsc.bitcast(jnp.int32(0x5F3759DF) - lax.shift_right_arithmetic(bi, 1),
                       jnp.float32)
      for _ in range(3):
        y = y * (1.5 - 0.5 * vv * y * y)
      for j in range(8):
        obuf[t, pl.ds(j * 16, 16)] = (cs[j] * y) * gv[j] + bv[j]
    return c

  def gather(i, buf, sem):
    return pltpu.make_async_copy(word_h.at[widx_v.at[pl.ds(i * T, T)]],
                                 buf, sem)

  def outcp(i, buf, sem):
    return pltpu.make_async_copy(buf, out_h.at[pl.ds(tok0 + i * T, T)], sem)

  def compute(i, wbuf, obuf):
    lax.fori_loop(0, T // 16, functools.partial(group_body, i, wbuf, obuf), 0)

  gather(0, wbuf0, gsem0).start()

  def pair(g, c):
    i0, i1 = 2 * g, 2 * g + 1
    gather(i0, wbuf0, gsem0).wait()
    gather(i1, wbuf1, gsem1).start()
    pl.when(g > 0)(lambda: outcp(i0, obuf0, osem0).wait())
    compute(i0, wbuf0, obuf0)
    outcp(i0, obuf0, osem0).start()
    pl.when(i1 + 1 < NSTEPS)(lambda: gather(i1 + 1, wbuf0, gsem0).start())
    gather(i1, wbuf1, gsem1).wait()
    pl.when(g > 0)(lambda: outcp(i1, obuf1, osem1).wait())
    compute(i1, wbuf1, obuf1)
    outcp(i1, obuf1, osem1).start()
    return c

  lax.fori_loop(0, NSTEPS // 2, pair, 0)
  outcp(NSTEPS - 2, obuf0, osem0).wait()
  outcp(NSTEPS - 1, obuf1, osem1).wait()


_emb = functools.partial(
    pl.kernel,
    out_type=jax.ShapeDtypeStruct((N_TOK, HID), jnp.float32),
    mesh=plsc.VectorSubcoreMesh(core_axis_name="c", subcore_axis_name="s",
                                num_cores=NC, num_subcores=NS),
    compiler_params=pltpu.CompilerParams(needs_layout_passes=False),
    scratch_types=[
        pltpu.VMEM((CHUNK,), jnp.int32),
        pltpu.VMEM((CHUNK,), jnp.int32),
        pltpu.VMEM((CHUNK,), jnp.int32),
        pltpu.VMEM((SEQ * HID,), jnp.float32),
        pltpu.VMEM((51 * HID,), jnp.float32),
        pltpu.VMEM((101 * HID,), jnp.float32),
        pltpu.VMEM((HID,), jnp.float32),
        pltpu.VMEM((HID,), jnp.float32),
        pltpu.VMEM((HID,), jnp.float32),
        pltpu.VMEM((T, HID), jnp.float32),
        pltpu.VMEM((T, HID), jnp.float32),
        pltpu.VMEM((T, HID), jnp.float32),
        pltpu.VMEM((T, HID), jnp.float32),
        pltpu.SemaphoreType.DMA,
        pltpu.SemaphoreType.DMA,
        pltpu.SemaphoreType.DMA,
        pltpu.SemaphoreType.DMA,
    ],
)(_body)


def kernel(input_ids, column_ids, row_ids, word_emb, pos_emb, type_emb,
           col_emb, row_emb, ln_gamma, ln_beta):
  bsz, seq_len = input_ids.shape
  ids = input_ids.reshape(-1).astype(jnp.int32)
  cids = column_ids.reshape(-1).astype(jnp.int32)
  rids = row_ids.reshape(-1).astype(jnp.int32)
  out = _emb(ids, cids, rids, word_emb, pos_emb.reshape(-1),
             type_emb[0], col_emb.reshape(-1), row_emb.reshape(-1),
             ln_gamma, ln_beta)
  return out.reshape(bsz, seq_len, HID)


# speedup vs baseline: 1.6039x; 1.5940x over previous
import functools

import jax
import jax.numpy as jnp
from jax import lax
from jax.experimental import pallas as pl
from jax.experimental.pallas import tpu as pltpu
from jax.experimental.pallas import tpu_sc as plsc

HID = 128
SEQ = 200
N_TOK = 1024 * 200
NC, NS = 2, 16
NW = NC * NS
CHUNK = N_TOK // NW
T = 128
NSTEPS = CHUNK // T
EPS = 1e-12


def _body(ids_h, cids_h, rids_h, word_h, pos_h, type_h, col_h, row_h,
          gam_h, bet_h, out_h,
          widx_v, cidx_v, ridx_v, base_v, colt_v, rowt_v,
          typ_v, gam_v, bet_v, wbuf0, obuf0, gsem0, osem0):
  wid = lax.axis_index("s") * NC + lax.axis_index("c")
  tok0 = wid * CHUNK

  pltpu.sync_copy(ids_h.at[pl.ds(tok0, CHUNK)], widx_v)
  pltpu.sync_copy(cids_h.at[pl.ds(tok0, CHUNK)], cidx_v)
  pltpu.sync_copy(rids_h.at[pl.ds(tok0, CHUNK)], ridx_v)
  pltpu.sync_copy(pos_h.at[pl.ds(0, SEQ * HID)], base_v)
  pltpu.sync_copy(col_h, colt_v)
  pltpu.sync_copy(row_h, rowt_v)
  pltpu.sync_copy(type_h, typ_v)
  pltpu.sync_copy(gam_h, gam_v)
  pltpu.sync_copy(bet_h, bet_v)

  tv = [typ_v[pl.ds(j * 16, 16)] for j in range(8)]

  def fold_type(s, c):
    for j in range(8):
      off = s * HID + j * 16
      base_v[pl.ds(off, 16)] = base_v[pl.ds(off, 16)] + tv[j]
    return c
  lax.fori_loop(0, SEQ, fold_type, 0)

  gv = [gam_v[pl.ds(j * 16, 16)] for j in range(8)]
  bv = [bet_v[pl.ds(j * 16, 16)] for j in range(8)]
  iot = lax.iota(jnp.int32, 16)

  def group_body(i, wbuf, obuf, g, c):
    gbase = i * T + g * 16
    civ = cidx_v[pl.ds(gbase, 16)]
    riv = ridx_v[pl.ds(gbase, 16)]
    for k in range(16):
      t = g * 16 + k
      s = lax.rem(gbase + k, SEQ)
      cb = civ[k] * HID
      rb = riv[k] * HID
      sb = s * HID
      xs = []
      for j in range(8):
        w = wbuf[t, pl.ds(j * 16, 16)]
        b = base_v[pl.ds(sb + j * 16, 16)]
        cvec = plsc.load_gather(colt_v, [cb + j * 16 + iot])
        rvec = plsc.load_gather(rowt_v, [rb + j * 16 + iot])
        xs.append((w + b) + (cvec + rvec))
      acc = ((xs[0] + xs[1]) + (xs[2] + xs[3])) + ((xs[4] + xs[5]) + (xs[6] + xs[7]))
      mean = jnp.sum(acc) * (1.0 / HID)
      cs = [x - mean for x in xs]
      sq = (((cs[0] * cs[0] + cs[1] * cs[1]) + (cs[2] * cs[2] + cs[3] * cs[3]))
            + ((cs[4] * cs[4] + cs[5] * cs[5]) + (cs[6] * cs[6] + cs[7] * cs[7])))
      var = jnp.sum(sq) * (1.0 / HID)
      vv = jnp.broadcast_to(var + EPS, (16,))
      bi = plsc.bitcast(vv, jnp.int32)
      y = plsc.bitcast(jnp.int32(0x5F3759DF) - lax.shift_right_arithmetic(bi, 1),
                       jnp.float32)
      for _ in range(3):
        y = y * (1.5 - 0.5 * vv * y * y)
      for j in range(8):
        obuf[t, pl.ds(j * 16, 16)] = (cs[j] * y) * gv[j] + bv[j]
    return c

  def gather(i, buf, sem):
    return pltpu.make_async_copy(word_h.at[widx_v.at[pl.ds(i * T, T)]],
                                 buf, sem)

  def outcp(i, buf, sem):
    return pltpu.make_async_copy(buf, out_h.at[pl.ds(tok0 + i * T, T)], sem)

  def compute(i, wbuf, obuf):
    lax.fori_loop(0, T // 16, functools.partial(group_body, i, wbuf, obuf), 0)

  def step_sync(i, c):
    gather(i, wbuf0, gsem0).start()
    gather(i, wbuf0, gsem0).wait()
    compute(i, wbuf0, obuf0)
    outcp(i, obuf0, osem0).start()
    outcp(i, obuf0, osem0).wait()
    return c

  lax.fori_loop(0, NSTEPS, step_sync, 0)


_emb = functools.partial(
    pl.kernel,
    out_type=jax.ShapeDtypeStruct((N_TOK, HID), jnp.float32),
    mesh=plsc.VectorSubcoreMesh(core_axis_name="c", subcore_axis_name="s",
                                num_cores=NC, num_subcores=NS),
    compiler_params=pltpu.CompilerParams(needs_layout_passes=False),
    scratch_types=[
        pltpu.VMEM((CHUNK,), jnp.int32),
        pltpu.VMEM((CHUNK,), jnp.int32),
        pltpu.VMEM((CHUNK,), jnp.int32),
        pltpu.VMEM((SEQ * HID,), jnp.float32),
        pltpu.VMEM((51 * HID,), jnp.float32),
        pltpu.VMEM((101 * HID,), jnp.float32),
        pltpu.VMEM((HID,), jnp.float32),
        pltpu.VMEM((HID,), jnp.float32),
        pltpu.VMEM((HID,), jnp.float32),
        pltpu.VMEM((T, HID), jnp.float32),
        pltpu.VMEM((T, HID), jnp.float32),
        pltpu.SemaphoreType.DMA,
        pltpu.SemaphoreType.DMA,
    ],
)(_body)


def kernel(input_ids, column_ids, row_ids, word_emb, pos_emb, type_emb,
           col_emb, row_emb, ln_gamma, ln_beta):
  bsz, seq_len = input_ids.shape
  ids = input_ids.reshape(-1).astype(jnp.int32)
  cids = column_ids.reshape(-1).astype(jnp.int32)
  rids = row_ids.reshape(-1).astype(jnp.int32)
  out = _emb(ids, cids, rids, word_emb, pos_emb.reshape(-1),
             type_emb[0], col_emb.reshape(-1), row_emb.reshape(-1),
             ln_gamma, ln_beta)
  return out.reshape(bsz, seq_len, HID)
